# baseline (speedup 1.0000x reference)
import functools

import jax
import jax.numpy as jnp
from jax import lax
from jax.experimental import pallas as pl
from jax.experimental.pallas import tpu as pltpu
from jax.experimental.pallas import tpu_sc as plsc

EPS_LN = 1e-12

_W = 128
_BK = 2048
_RB = 1600


def _tp_body(a_ref, b_ref, o_ref):
    o_ref[:, 0:64] = a_ref[...].astype(jnp.bfloat16).T
    o_ref[:, 64:128] = b_ref[...].astype(jnp.bfloat16).T


def _transpose_pack(wt, v, h):
    grid = (v // 2 + _BK - 1) // _BK
    max_blk = (v + _BK - 1) // _BK - 1
    return pl.pallas_call(
        _tp_body,
        grid=(grid,),
        in_specs=[
            pl.BlockSpec((h, _BK), lambda i: (0, 2 * i)),
            pl.BlockSpec((h, _BK),
                         lambda i, m=max_blk: (0, jnp.minimum(2 * i + 1, m))),
        ],
        out_specs=pl.BlockSpec((_BK, 128), lambda i: (i, 0)),
        out_shape=jax.ShapeDtypeStruct((grid * _BK, 128), jnp.bfloat16),
    )(wt, wt)


def _sc_gather(table, idx2d, bl, h):
    mesh = plsc.VectorSubcoreMesh(core_axis_name="c", subcore_axis_name="s")

    @functools.partial(
        pl.kernel,
        out_type=jax.ShapeDtypeStruct((bl, h), jnp.bfloat16),
        mesh=mesh,
        compiler_params=pltpu.CompilerParams(use_tc_tiling_on_sc=False),
    )
    def gather_kernel(tbl_hbm, idx_hbm, out_hbm):
        def body(i_vmem, o_vmem):
            pltpu.sync_copy(tbl_hbm.at[i_vmem.at[0]], o_vmem)

        pltpu.emit_pipeline(
            body,
            grid=(bl // _W,),
            in_specs=[pl.BlockSpec((1, _W), lambda i: (0, i))],
            out_specs=[pl.BlockSpec((_W, h), lambda i: (i, 0))],
            core_axis_name=("c", "s"),
            dimension_semantics=(pltpu.PARALLEL,),
        )(idx_hbm, out_hbm)

    return gather_kernel(table, idx2d)


def _ln_body(x_ref, p_ref, g_ref, b_ref, o_ref):
    x = x_ref[...].astype(jnp.float32) + p_ref[...]
    lane = lax.broadcasted_iota(jnp.int32, x.shape, 1)
    mlo = (lane < 64).astype(jnp.float32)
    mhi = 1.0 - mlo
    slo = jnp.sum(x * mlo, axis=-1, keepdims=True)
    shi = jnp.sum(x * mhi, axis=-1, keepdims=True)
    mu = (slo * mlo + shi * mhi) * (1.0 / 64.0)
    xc = x - mu
    x2 = xc * xc
    vlo = jnp.sum(x2 * mlo, axis=-1, keepdims=True)
    vhi = jnp.sum(x2 * mhi, axis=-1, keepdims=True)
    var = (vlo * mlo + vhi * mhi) * (1.0 / 64.0)
    o_ref[...] = xc * lax.rsqrt(var + EPS_LN) * g_ref[...] + b_ref[...]


def _tc_ln(pairs, pos_full, gamma2, beta2):
    n = pairs.shape[0]
    return pl.pallas_call(
        _ln_body,
        grid=(n // _RB,),
        in_specs=[
            pl.BlockSpec((_RB, 128), lambda i: (i, 0)),
            pl.BlockSpec((_RB, 128), lambda i: (0, 0)),
            pl.BlockSpec((1, 128), lambda i: (0, 0)),
            pl.BlockSpec((1, 128), lambda i: (0, 0)),
        ],
        out_specs=pl.BlockSpec((_RB, 128), lambda i: (i, 0)),
        out_shape=jax.ShapeDtypeStruct((n, 128), jnp.float32),
    )(pairs, pos_full, gamma2, beta2)


def kernel(prefix_text, word_embeddings, position_embeddings, ln_gamma, ln_beta):
    b, l = prefix_text.shape
    v, h = word_embeddings.shape
    bl = b * l

    packed = _transpose_pack(word_embeddings.T, v, h)
    table = packed.reshape(2 * packed.shape[0], h)

    idx = prefix_text.astype(jnp.int32).reshape(1, bl)
    blk = idx // (2 * _BK)
    j = idx - 2 * _BK * blk
    idx2d = jnp.where(j < _BK,
                      2 * (_BK * blk + j),
                      2 * (_BK * blk + j - _BK) + 1)
    gathered = _sc_gather(table, idx2d, bl, h)

    pairs = gathered.reshape(bl // 2, 2 * h)
    pos_full = jnp.tile(position_embeddings[:l].reshape(l // 2, 2 * h),
                        (_RB // (l // 2), 1))
    gamma2 = jnp.tile(ln_gamma, 2).reshape(1, 2 * h)
    beta2 = jnp.tile(ln_beta, 2).reshape(1, 2 * h)
    out = _tc_ln(pairs, pos_full, gamma2, beta2)
    return out.reshape(b, l, h)


# speedup vs baseline: 1.3762x; 1.3762x over previous
import functools

import jax
import jax.numpy as jnp
from jax import lax
from jax.experimental import pallas as pl
from jax.experimental.pallas import tpu as pltpu
from jax.experimental.pallas import tpu_sc as plsc

EPS_LN = 1e-12

_W = 128
_BK = 4096
_RB = 1600


def _tp_body(a_ref, b_ref, o_ref):
    o_ref[:, 0:64] = a_ref[...].T
    o_ref[:, 64:128] = b_ref[...].T


def _transpose_pack(wt, v, h):
    grid = (v // 2 + _BK - 1) // _BK
    max_blk = (v + _BK - 1) // _BK - 1
    return pl.pallas_call(
        _tp_body,
        grid=(grid,),
        in_specs=[
            pl.BlockSpec((h, _BK), lambda i: (0, 2 * i)),
            pl.BlockSpec((h, _BK),
                         lambda i, m=max_blk: (0, jnp.minimum(2 * i + 1, m))),
        ],
        out_specs=pl.BlockSpec((_BK, 128), lambda i: (i, 0)),
        out_shape=jax.ShapeDtypeStruct((grid * _BK, 128), jnp.float32),
    )(wt, wt)


def _sc_gather(table, idx2d, n, h):
    mesh = plsc.VectorSubcoreMesh(core_axis_name="c", subcore_axis_name="s")

    @functools.partial(
        pl.kernel,
        out_type=jax.ShapeDtypeStruct((n, h), jnp.float32),
        mesh=mesh,
        compiler_params=pltpu.CompilerParams(use_tc_tiling_on_sc=False),
    )
    def gather_kernel(tbl_hbm, idx_hbm, out_hbm):
        def body(i_vmem, o_vmem):
            pltpu.sync_copy(tbl_hbm.at[i_vmem.at[0]], o_vmem)

        pltpu.emit_pipeline(
            body,
            grid=(n // _W,),
            in_specs=[pl.BlockSpec((1, _W), lambda i: (0, i))],
            out_specs=[pl.BlockSpec((_W, h), lambda i: (i, 0))],
            core_axis_name=("c", "s"),
            dimension_semantics=(pltpu.PARALLEL,),
        )(idx_hbm, out_hbm)

    return gather_kernel(table, idx2d)


def _ln_body(x_ref, p_ref, g_ref, b_ref, o_ref):
    x = x_ref[...] + p_ref[...]
    lane = lax.broadcasted_iota(jnp.int32, x.shape, 1)
    mlo = (lane < 64).astype(jnp.float32)
    mu = jnp.sum(x * mlo, axis=-1, keepdims=True) * (1.0 / 64.0)
    xc = x - mu
    var = jnp.sum(xc * xc * mlo, axis=-1, keepdims=True) * (1.0 / 64.0)
    y = xc * lax.rsqrt(var + EPS_LN) * g_ref[...] + b_ref[...]
    o_ref[...] = y[:, 0:64].reshape(o_ref.shape)


def _tc_ln(dup, pos_dup, gamma, beta, b, l, h):
    n = dup.shape[0]
    bb = _RB // l
    return pl.pallas_call(
        _ln_body,
        grid=(n // _RB,),
        in_specs=[
            pl.BlockSpec((_RB, 128), lambda i: (i, 0)),
            pl.BlockSpec((_RB, 128), lambda i: (0, 0)),
            pl.BlockSpec((1, 128), lambda i: (0, 0)),
            pl.BlockSpec((1, 128), lambda i: (0, 0)),
        ],
        out_specs=pl.BlockSpec((bb, l, h), lambda i: (i, 0, 0)),
        out_shape=jax.ShapeDtypeStruct((b, l, h), jnp.float32),
    )(dup, pos_dup, gamma, beta)


def kernel(prefix_text, word_embeddings, position_embeddings, ln_gamma, ln_beta):
    b, l = prefix_text.shape
    v, h = word_embeddings.shape
    bl = b * l

    packed = _transpose_pack(word_embeddings.T, v, h)
    table = packed.reshape(2 * packed.shape[0], h)

    idx = prefix_text.astype(jnp.int32).reshape(1, bl)
    blk = idx // (2 * _BK)
    j = idx - 2 * _BK * blk
    q = jnp.where(j < _BK,
                  2 * (_BK * blk + j),
                  2 * (_BK * blk + j - _BK) + 1)
    qq = jnp.repeat(q, 2, axis=1)
    dup = _sc_gather(table, qq, 2 * bl, h)
    dup = dup.reshape(bl, 2 * h)

    pos200 = position_embeddings[:l]
    pos_dup = jnp.tile(jnp.concatenate([pos200, pos200], axis=1),
                       (_RB // l, 1))
    gamma2 = jnp.tile(ln_gamma, 2).reshape(1, 2 * h)
    beta2 = jnp.tile(ln_beta, 2).reshape(1, 2 * h)
    return _tc_ln(dup, pos_dup, gamma2, beta2, b, l, h)


# speedup vs baseline: 1.9153x; 1.3917x over previous
import functools

import jax
import jax.numpy as jnp
from jax import lax
from jax.experimental import pallas as pl
from jax.experimental.pallas import tpu as pltpu
from jax.experimental.pallas import tpu_sc as plsc

EPS_LN = 1e-12

_W = 128
_BK = 4096
_RB = 1600


def _tp_body(a_ref, b_ref, o_ref):
    o_ref[:, 0:64] = a_ref[...].T
    o_ref[:, 64:128] = b_ref[...].T


def _transpose_pack(wt, v, h):
    grid = (v // 2 + _BK - 1) // _BK
    max_blk = (v + _BK - 1) // _BK - 1
    return pl.pallas_call(
        _tp_body,
        grid=(grid,),
        in_specs=[
            pl.BlockSpec((h, _BK), lambda i: (0, 2 * i)),
            pl.BlockSpec((h, _BK),
                         lambda i, m=max_blk: (0, jnp.minimum(2 * i + 1, m))),
        ],
        out_specs=pl.BlockSpec((_BK, 128), lambda i: (i, 0)),
        out_shape=jax.ShapeDtypeStruct((grid * _BK, 128), jnp.float32),
    )(wt, wt)


def _sc_gather(table, idx2d, bl, h):
    mesh = plsc.VectorSubcoreMesh(core_axis_name="c", subcore_axis_name="s")

    @functools.partial(
        pl.kernel,
        out_type=jax.ShapeDtypeStruct((bl, h), jnp.float32),
        mesh=mesh,
        compiler_params=pltpu.CompilerParams(use_tc_tiling_on_sc=False),
    )
    def gather_kernel(tbl_hbm, idx_hbm, out_hbm):
        def body(i_vmem, o_vmem):
            pltpu.sync_copy(tbl_hbm.at[i_vmem.at[0]], o_vmem)

        pltpu.emit_pipeline(
            body,
            grid=(bl // _W,),
            in_specs=[pl.BlockSpec((1, _W), lambda i: (0, i))],
            out_specs=[pl.BlockSpec((_W, h), lambda i: (i, 0))],
            core_axis_name=("c", "s"),
            dimension_semantics=(pltpu.PARALLEL,),
        )(idx_hbm, out_hbm)

    return gather_kernel(table, idx2d)


def _ln_body(x_ref, p_ref, g_ref, b_ref, o_ref):
    x = x_ref[...] + p_ref[...]
    lane = lax.broadcasted_iota(jnp.int32, x.shape, 1)
    mlo = (lane < 64).astype(jnp.float32)
    mhi = 1.0 - mlo
    slo = jnp.sum(x * mlo, axis=-1, keepdims=True)
    shi = jnp.sum(x * mhi, axis=-1, keepdims=True)
    mu = (slo * mlo + shi * mhi) * (1.0 / 64.0)
    xc = x - mu
    x2 = xc * xc
    vlo = jnp.sum(x2 * mlo, axis=-1, keepdims=True)
    vhi = jnp.sum(x2 * mhi, axis=-1, keepdims=True)
    var = (vlo * mlo + vhi * mhi) * (1.0 / 64.0)
    o_ref[...] = xc * lax.rsqrt(var + EPS_LN) * g_ref[...] + b_ref[...]


def _tc_ln(pairs, pos_full, gamma2, beta2):
    n = pairs.shape[0]
    return pl.pallas_call(
        _ln_body,
        grid=(n // _RB,),
        in_specs=[
            pl.BlockSpec((_RB, 128), lambda i: (i, 0)),
            pl.BlockSpec((_RB, 128), lambda i: (0, 0)),
            pl.BlockSpec((1, 128), lambda i: (0, 0)),
            pl.BlockSpec((1, 128), lambda i: (0, 0)),
        ],
        out_specs=pl.BlockSpec((_RB, 128), lambda i: (i, 0)),
        out_shape=jax.ShapeDtypeStruct((n, 128), jnp.float32),
    )(pairs, pos_full, gamma2, beta2)


def kernel(prefix_text, word_embeddings, position_embeddings, ln_gamma, ln_beta):
    b, l = prefix_text.shape
    v, h = word_embeddings.shape
    bl = b * l

    packed = _transpose_pack(word_embeddings.T, v, h)
    table = packed.reshape(2 * packed.shape[0], h)

    idx = prefix_text.astype(jnp.int32).reshape(1, bl)
    blk = idx // (2 * _BK)
    j = idx - 2 * _BK * blk
    idx2d = jnp.where(j < _BK,
                      2 * (_BK * blk + j),
                      2 * (_BK * blk + j - _BK) + 1)
    gathered = _sc_gather(table, idx2d, bl, h)

    pairs = gathered.reshape(bl // 2, 2 * h)
    pos_full = jnp.tile(position_embeddings[:l].reshape(l // 2, 2 * h),
                        (_RB // (l // 2), 1))
    gamma2 = jnp.tile(ln_gamma, 2).reshape(1, 2 * h)
    beta2 = jnp.tile(ln_beta, 2).reshape(1, 2 * h)
    out = _tc_ln(pairs, pos_full, gamma2, beta2)
    return out.reshape(b, l, h)


# speedup vs baseline: 2.0915x; 1.0920x over previous
import functools

import jax
import jax.numpy as jnp
from jax import lax
from jax.experimental import pallas as pl
from jax.experimental.pallas import tpu as pltpu
from jax.experimental.pallas import tpu_sc as plsc

EPS_LN = 1e-12

_W = 128
_BK = 8192
_RB = 3200


def _tp_body(a_ref, b_ref, o_ref):
    o_ref[:, 0:64] = a_ref[...].T
    o_ref[:, 64:128] = b_ref[...].T


def _transpose_pack(wt, v, h):
    grid = (v // 2 + _BK - 1) // _BK
    max_blk = (v + _BK - 1) // _BK - 1
    return pl.pallas_call(
        _tp_body,
        grid=(grid,),
        in_specs=[
            pl.BlockSpec((h, _BK), lambda i: (0, 2 * i)),
            pl.BlockSpec((h, _BK),
                         lambda i, m=max_blk: (0, jnp.minimum(2 * i + 1, m))),
        ],
        out_specs=pl.BlockSpec((_BK, 128), lambda i: (i, 0)),
        out_shape=jax.ShapeDtypeStruct((grid * _BK, 128), jnp.float32),
    )(wt, wt)


def _sc_gather(table, idx2d, bl, h):
    mesh = plsc.VectorSubcoreMesh(core_axis_name="c", subcore_axis_name="s")

    @functools.partial(
        pl.kernel,
        out_type=jax.ShapeDtypeStruct((bl, h), jnp.float32),
        mesh=mesh,
        compiler_params=pltpu.CompilerParams(use_tc_tiling_on_sc=False),
    )
    def gather_kernel(tbl_hbm, idx_hbm, out_hbm):
        def body(i_vmem, o_vmem):
            pltpu.sync_copy(tbl_hbm.at[i_vmem.at[0]], o_vmem)

        pltpu.emit_pipeline(
            body,
            grid=(bl // _W,),
            in_specs=[pl.BlockSpec((1, _W), lambda i: (0, i))],
            out_specs=[pl.BlockSpec((_W, h), lambda i: (i, 0))],
            core_axis_name=("c", "s"),
            dimension_semantics=(pltpu.PARALLEL,),
        )(idx_hbm, out_hbm)

    return gather_kernel(table, idx2d)


def _ln_body(x_ref, p_ref, g_ref, b_ref, o_ref):
    x = x_ref[...] + p_ref[...]
    lane = lax.broadcasted_iota(jnp.int32, x.shape, 1)
    mlo = (lane < 64).astype(jnp.float32)
    mhi = 1.0 - mlo
    slo = jnp.sum(x * mlo, axis=-1, keepdims=True)
    shi = jnp.sum(x * mhi, axis=-1, keepdims=True)
    mu = (slo * mlo + shi * mhi) * (1.0 / 64.0)
    xc = x - mu
    x2 = xc * xc
    vlo = jnp.sum(x2 * mlo, axis=-1, keepdims=True)
    vhi = jnp.sum(x2 * mhi, axis=-1, keepdims=True)
    var = (vlo * mlo + vhi * mhi) * (1.0 / 64.0)
    o_ref[...] = xc * lax.rsqrt(var + EPS_LN) * g_ref[...] + b_ref[...]


def _tc_ln(pairs, pos_full, gamma2, beta2):
    n = pairs.shape[0]
    return pl.pallas_call(
        _ln_body,
        grid=(n // _RB,),
        in_specs=[
            pl.BlockSpec((_RB, 128), lambda i: (i, 0)),
            pl.BlockSpec((_RB, 128), lambda i: (0, 0)),
            pl.BlockSpec((1, 128), lambda i: (0, 0)),
            pl.BlockSpec((1, 128), lambda i: (0, 0)),
        ],
        out_specs=pl.BlockSpec((_RB, 128), lambda i: (i, 0)),
        out_shape=jax.ShapeDtypeStruct((n, 128), jnp.float32),
    )(pairs, pos_full, gamma2, beta2)


def kernel(prefix_text, word_embeddings, position_embeddings, ln_gamma, ln_beta):
    b, l = prefix_text.shape
    v, h = word_embeddings.shape
    bl = b * l

    packed = _transpose_pack(word_embeddings.T, v, h)
    table = packed.reshape(2 * packed.shape[0], h)

    idx = prefix_text.astype(jnp.int32).reshape(1, bl)
    blk = idx // (2 * _BK)
    j = idx - 2 * _BK * blk
    idx2d = jnp.where(j < _BK,
                      2 * (_BK * blk + j),
                      2 * (_BK * blk + j - _BK) + 1)
    gathered = _sc_gather(table, idx2d, bl, h)

    pairs = gathered.reshape(bl // 2, 2 * h)
    pos_full = jnp.tile(position_embeddings[:l].reshape(l // 2, 2 * h),
                        (_RB // (l // 2), 1))
    gamma2 = jnp.tile(ln_gamma, 2).reshape(1, 2 * h)
    beta2 = jnp.tile(ln_beta, 2).reshape(1, 2 * h)
    out = _tc_ln(pairs, pos_full, gamma2, beta2)
    return out.reshape(b, l, h)


# speedup vs baseline: 2.1637x; 1.0346x over previous
import functools

import jax
import jax.numpy as jnp
from jax import lax
from jax.experimental import pallas as pl
from jax.experimental.pallas import tpu as pltpu
from jax.experimental.pallas import tpu_sc as plsc

EPS_LN = 1e-12

_W = 128
_BK = 16384
_RB = 6400


def _tp_body(a_ref, b_ref, o_ref):
    o_ref[:, 0:64] = a_ref[...].T
    o_ref[:, 64:128] = b_ref[...].T


def _transpose_pack(wt, v, h):
    grid = (v // 2 + _BK - 1) // _BK
    max_blk = (v + _BK - 1) // _BK - 1
    return pl.pallas_call(
        _tp_body,
        grid=(grid,),
        in_specs=[
            pl.BlockSpec((h, _BK), lambda i: (0, 2 * i)),
            pl.BlockSpec((h, _BK),
                         lambda i, m=max_blk: (0, jnp.minimum(2 * i + 1, m))),
        ],
        out_specs=pl.BlockSpec((_BK, 128), lambda i: (i, 0)),
        out_shape=jax.ShapeDtypeStruct((grid * _BK, 128), jnp.float32),
    )(wt, wt)


def _sc_gather(table, idx2d, bl, h):
    mesh = plsc.VectorSubcoreMesh(core_axis_name="c", subcore_axis_name="s")

    @functools.partial(
        pl.kernel,
        out_type=jax.ShapeDtypeStruct((bl, h), jnp.float32),
        mesh=mesh,
        compiler_params=pltpu.CompilerParams(use_tc_tiling_on_sc=False),
    )
    def gather_kernel(tbl_hbm, idx_hbm, out_hbm):
        def body(i_vmem, o_vmem):
            pltpu.sync_copy(tbl_hbm.at[i_vmem.at[0]], o_vmem)

        pltpu.emit_pipeline(
            body,
            grid=(bl // _W,),
            in_specs=[pl.BlockSpec((1, _W), lambda i: (0, i))],
            out_specs=[pl.BlockSpec((_W, h), lambda i: (i, 0))],
            core_axis_name=("c", "s"),
            dimension_semantics=(pltpu.PARALLEL,),
        )(idx_hbm, out_hbm)

    return gather_kernel(table, idx2d)


def _ln_body(x_ref, p_ref, g_ref, b_ref, o_ref):
    x = x_ref[...] + p_ref[...]
    lane = lax.broadcasted_iota(jnp.int32, x.shape, 1)
    mlo = (lane < 64).astype(jnp.float32)
    mhi = 1.0 - mlo
    slo = jnp.sum(x * mlo, axis=-1, keepdims=True)
    shi = jnp.sum(x * mhi, axis=-1, keepdims=True)
    mu = (slo * mlo + shi * mhi) * (1.0 / 64.0)
    xc = x - mu
    x2 = xc * xc
    vlo = jnp.sum(x2 * mlo, axis=-1, keepdims=True)
    vhi = jnp.sum(x2 * mhi, axis=-1, keepdims=True)
    var = (vlo * mlo + vhi * mhi) * (1.0 / 64.0)
    o_ref[...] = xc * lax.rsqrt(var + EPS_LN) * g_ref[...] + b_ref[...]


def _tc_ln(pairs, pos_full, gamma2, beta2):
    n = pairs.shape[0]
    return pl.pallas_call(
        _ln_body,
        grid=(n // _RB,),
        in_specs=[
            pl.BlockSpec((_RB, 128), lambda i: (i, 0)),
            pl.BlockSpec((_RB, 128), lambda i: (0, 0)),
            pl.BlockSpec((1, 128), lambda i: (0, 0)),
            pl.BlockSpec((1, 128), lambda i: (0, 0)),
        ],
        out_specs=pl.BlockSpec((_RB, 128), lambda i: (i, 0)),
        out_shape=jax.ShapeDtypeStruct((n, 128), jnp.float32),
    )(pairs, pos_full, gamma2, beta2)


def kernel(prefix_text, word_embeddings, position_embeddings, ln_gamma, ln_beta):
    b, l = prefix_text.shape
    v, h = word_embeddings.shape
    bl = b * l

    packed = _transpose_pack(word_embeddings.T, v, h)
    table = packed.reshape(2 * packed.shape[0], h)

    idx = prefix_text.astype(jnp.int32).reshape(1, bl)
    blk = idx // (2 * _BK)
    j = idx - 2 * _BK * blk
    idx2d = jnp.where(j < _BK,
                      2 * (_BK * blk + j),
                      2 * (_BK * blk + j - _BK) + 1)
    gathered = _sc_gather(table, idx2d, bl, h)

    pairs = gathered.reshape(bl // 2, 2 * h)
    pos_full = jnp.tile(position_embeddings[:l].reshape(l // 2, 2 * h),
                        (_RB // (l // 2), 1))
    gamma2 = jnp.tile(ln_gamma, 2).reshape(1, 2 * h)
    beta2 = jnp.tile(ln_beta, 2).reshape(1, 2 * h)
    out = _tc_ln(pairs, pos_full, gamma2, beta2)
    return out.reshape(b, l, h)
